# baseline (speedup 1.0000x reference)
import functools

import jax
import jax.numpy as jnp
from jax import lax
from jax.experimental import pallas as pl
from jax.experimental.pallas import tpu as pltpu
from jax.experimental.pallas import tpu_sc as plsc

_LAMBDA = 10.0
_NC = 2
_NS = 16
_NW = _NC * _NS
_L = 16


def _make_sc_loss(B, V, SPAD, R, UN):
    assert B % (_NW * R) == 0 and R % _L == 0 and V % UN == 0
    RPW = B // _NW
    NCHUNK = RPW // R
    NGROUP = R // _L

    mesh = plsc.VectorSubcoreMesh(core_axis_name="c", subcore_axis_name="s")

    @functools.partial(
        pl.kernel,
        mesh=mesh,
        out_type=jax.ShapeDtypeStruct((_NW, _L), jnp.float32),
        compiler_params=pltpu.CompilerParams(needs_layout_passes=False),
        scratch_types=[
            pltpu.VMEM((R * V,), jnp.float32),
            pltpu.VMEM((R * SPAD,), jnp.int32),
            pltpu.VMEM((RPW,), jnp.float32),
            pltpu.VMEM((_L * V,), jnp.int32),
            pltpu.VMEM((_L,), jnp.float32),
        ],
    )
    def sc_loss(pred_hbm, tgt_hbm, w_hbm, out_hbm, pred_v, tgt_v, w_v, mark_v, out_v):
        wid = lax.axis_index("s") * _NC + lax.axis_index("c")
        row0 = wid * RPW
        lanes = lax.iota(jnp.int32, _L)
        lane_v = lanes * V

        neg1 = jnp.full((_L,), -1, jnp.int32)

        def init_body(k, _):
            mark_v[pl.ds(k * _L, _L)] = neg1
            return 0

        lax.fori_loop(0, V, init_body, 0)

        pltpu.sync_copy(w_hbm.at[pl.ds(row0, RPW)], w_v)

        zero = jnp.zeros((_L,), jnp.float32)

        def chunk_body(gidx, acc):
            base = row0 + gidx * R
            pltpu.sync_copy(pred_hbm.at[pl.ds(base * V, R * V)], pred_v)
            pltpu.sync_copy(tgt_hbm.at[pl.ds(base * SPAD, R * SPAD)], tgt_v)

            def group_body(gg, acc2):
                macc, sacc = acc2
                goff = gg * _L
                rows = goff + lanes
                pbase = rows * V
                tbase = rows * SPAD

                def dense_body(kk, dacc):
                    s2v, sev, idxv = dacc
                    for _ in range(UN):
                        p = plsc.load_gather(pred_v, [idxv])
                        s2v = s2v + p * p
                        sev = sev + jnp.exp(p)
                        idxv = idxv + 1
                    return (s2v, sev, idxv)

                s2v, sev, _ = lax.fori_loop(
                    0, V // UN, dense_body, (zero, zero, pbase)
                )

                mrow = (gidx * R + rows) * SPAD
                for j in range(SPAD):
                    t = plsc.load_gather(tgt_v, [tbase + j])
                    plsc.store_scatter(mark_v, [lane_v + t], mrow + j)
                npos_v = zero
                sg_v = zero
                spe_v = zero
                sne_v = zero
                for j in range(SPAD):
                    t = plsc.load_gather(tgt_v, [tbase + j])
                    rb = plsc.load_gather(mark_v, [lane_v + t])
                    keep = jnp.where(rb == mrow + j, 1.0, 0.0)
                    g = plsc.load_gather(pred_v, [pbase + t])
                    npos_v = npos_v + keep
                    sg_v = sg_v + keep * g
                    spe_v = spe_v + keep * jnp.exp(-g)
                    sne_v = sne_v + keep * jnp.exp(g)

                w16 = w_v[pl.ds(gidx * R + goff, _L)]
                mse_rv = w16 * (s2v - 2.0 * sg_v + npos_v)
                set_rv = spe_v * (sev - sne_v) / (npos_v * (float(V) - npos_v))
                return (macc + mse_rv, sacc + set_rv)

            return lax.fori_loop(0, NGROUP, group_body, acc)

        macc, sacc = lax.fori_loop(0, NCHUNK, chunk_body, (zero, zero))
        out_v[...] = macc * (1.0 / B) + (_LAMBDA / B) * sacc
        pltpu.sync_copy(out_v, out_hbm.at[wid])

    return sc_loss


def kernel(pred, target, weights):
    B, V = pred.shape
    S = target.shape[1]
    SPAD = 64
    tgt = jnp.concatenate(
        [target, jnp.broadcast_to(target[:, :1], (B, SPAD - S))], axis=1
    )
    fn = _make_sc_loss(B, V, SPAD, R=32, UN=8)
    partials = fn(pred.reshape(-1), tgt.reshape(-1), weights)
    return jnp.sum(partials)


# speedup vs baseline: 2.0678x; 2.0678x over previous
import functools

import jax
import jax.numpy as jnp
from jax import lax
from jax.experimental import pallas as pl
from jax.experimental.pallas import tpu as pltpu
from jax.experimental.pallas import tpu_sc as plsc

_LAMBDA = 10.0
_NC = 2
_NS = 16
_NW = _NC * _NS
_L = 16


def _make_sc_loss(B, V, SPAD, R):
    assert B % (_NW * 2 * R) == 0 and V % 8 == 0
    RPW = B // _NW
    NCHUNK = RPW // R
    NFULL = V // _L
    REM = V - NFULL * _L
    TAIL = V - _L
    NJ = SPAD // _L

    mesh = plsc.VectorSubcoreMesh(core_axis_name="c", subcore_axis_name="s")

    @functools.partial(
        pl.kernel,
        mesh=mesh,
        out_type=jax.ShapeDtypeStruct((_NW, _L), jnp.float32),
        compiler_params=pltpu.CompilerParams(needs_layout_passes=False),
        scratch_types=[
            pltpu.VMEM((2 * R, V), jnp.float32),
            pltpu.VMEM((2 * R, SPAD), jnp.int32),
            pltpu.VMEM((RPW + _L,), jnp.float32),
            pltpu.VMEM((V + 8,), jnp.int32),
            pltpu.VMEM((_L,), jnp.float32),
            pltpu.SemaphoreType.DMA,
            pltpu.SemaphoreType.DMA,
        ],
    )
    def sc_loss(pred_hbm, tgt_hbm, w_hbm, out_hbm,
                pred_v, tgt_v, w_v, mark_v, out_v, sem_a, sem_b):
        wid = lax.axis_index("s") * _NC + lax.axis_index("c")
        row0 = wid * RPW
        lanes = lax.iota(jnp.int32, _L)
        tailf = (jnp.where(lanes >= (_L - REM), 1.0, 0.0)
                 if REM else jnp.zeros((_L,), jnp.float32))

        def start_chunk(g, boff, sem):
            base = row0 + g * R
            pltpu.async_copy(pred_hbm.at[pl.ds(base, R), :],
                             pred_v.at[pl.ds(boff, R), :], sem)
            pltpu.async_copy(tgt_hbm.at[pl.ds(base, R), :],
                             tgt_v.at[pl.ds(boff, R), :], sem)

        def wait_chunk(boff, sem):
            pltpu.make_async_copy(pred_hbm.at[pl.ds(0, R), :],
                                  pred_v.at[pl.ds(boff, R), :], sem).wait()
            pltpu.make_async_copy(tgt_hbm.at[pl.ds(0, R), :],
                                  tgt_v.at[pl.ds(boff, R), :], sem).wait()

        neg1 = jnp.full((_L,), -1, jnp.int32)

        start_chunk(0, 0, sem_a)
        start_chunk(1, R, sem_b)

        def init_body(k, _):
            mark_v[pl.ds(k * _L, _L)] = neg1
            return 0

        lax.fori_loop(0, (V + 8) // _L, init_body, 0)

        pltpu.sync_copy(w_hbm.at[pl.ds(row0, RPW)], w_v.at[pl.ds(0, RPW)])

        def process(g, boff, sem, acc):
            wait_chunk(boff, sem)

            def row_body(r, acc2):
                mse_a, set_a = acc2

                zero = jnp.zeros((_L,), jnp.float32)
                s2v = [zero, zero]
                sev = [zero, zero]
                for k in range(NFULL):
                    p = pred_v[boff + r, pl.ds(k * _L, _L)]
                    s2v[k % 2] = s2v[k % 2] + p * p
                    sev[k % 2] = sev[k % 2] + jnp.exp(p)
                pt = pred_v[boff + r, pl.ds(TAIL, _L)]
                s2v_t = s2v[0] + s2v[1] + tailf * pt * pt
                sev_t = sev[0] + sev[1] + tailf * jnp.exp(pt)

                mbase = (g * R + r) * SPAD
                for j in range(NJ):
                    t = tgt_v[boff + r, pl.ds(j * _L, _L)]
                    plsc.store_scatter(mark_v, [t], mbase + j * _L + lanes)
                npos_i = jnp.zeros((_L,), jnp.int32)
                sg_v = zero
                spe_v = zero
                sne_v = zero
                rfull = jnp.full((_L,), boff + r, jnp.int32)
                for j in range(NJ):
                    t = tgt_v[boff + r, pl.ds(j * _L, _L)]
                    rb = plsc.load_gather(mark_v, [t])
                    kb = rb == mbase + j * _L + lanes
                    keep = jnp.where(kb, 1.0, 0.0)
                    g_j = plsc.load_gather(pred_v, [rfull, t])
                    npos_i = npos_i + plsc.all_reduce_population_count(kb)
                    sg_v = sg_v + keep * g_j
                    spe_v = spe_v + keep * jnp.exp(-g_j)
                    sne_v = sne_v + keep * jnp.exp(g_j)
                npos = npos_i.astype(jnp.float32)
                bv = lambda x: jnp.full((_L,), x, jnp.float32)
                mseg = bv(jnp.sum(s2v_t - 2.0 * sg_v))
                se = bv(jnp.sum(sev_t))
                spe = bv(jnp.sum(spe_v))
                sne = bv(jnp.sum(sne_v))
                w = bv(w_v[pl.ds(g * R + r, _L)][0])
                mse_r = w * (mseg + npos)
                set_r = spe * (se - sne) / (npos * (float(V) - npos))
                return (mse_a + mse_r, set_a + set_r)

            acc = lax.fori_loop(0, R, row_body, acc, unroll=2)

            @pl.when(g + 2 < NCHUNK)
            def _():
                start_chunk(g + 2, boff, sem)

            return acc

        def pair_body(i, acc):
            g = i * 2
            acc = process(g, 0, sem_a, acc)
            acc = process(g + 1, R, sem_b, acc)
            return acc

        zv = jnp.zeros((_L,), jnp.float32)
        mse_acc, set_acc = lax.fori_loop(0, NCHUNK // 2, pair_body, (zv, zv))
        total = mse_acc * (1.0 / B) + (_LAMBDA / B) * set_acc
        out_v[...] = jnp.where(lanes == 0, total, 0.0)
        pltpu.sync_copy(out_v, out_hbm.at[wid])

    return sc_loss


def kernel(pred, target, weights):
    B, V = pred.shape
    S = target.shape[1]
    SPAD = 64
    tgt = jnp.concatenate(
        [target, jnp.broadcast_to(target[:, :1], (B, SPAD - S))], axis=1
    )
    partials = _make_sc_loss(B, V, SPAD, R=32)(pred, tgt, weights)
    return jnp.sum(partials)


# speedup vs baseline: 2.2375x; 1.0821x over previous
import functools

import jax
import jax.numpy as jnp
from jax import lax
from jax.experimental import pallas as pl
from jax.experimental.pallas import tpu as pltpu
from jax.experimental.pallas import tpu_sc as plsc

_LAMBDA = 10.0
_NC = 2
_NS = 16
_NW = _NC * _NS
_L = 16


def _make_sc_loss(B, V, S, R):
    assert B % (_NW * 2 * R) == 0 and V % 8 == 0 and S >= _L
    RPW = B // _NW
    NCHUNK = RPW // R
    NFULL = V // _L
    REM = V - NFULL * _L
    TAIL = V - _L
    OFFS = [k * _L for k in range(S // _L)] + ([S - _L] if S % _L else [])
    NJ = len(OFFS)

    mesh = plsc.VectorSubcoreMesh(core_axis_name="c", subcore_axis_name="s")

    @functools.partial(
        pl.kernel,
        mesh=mesh,
        out_type=jax.ShapeDtypeStruct((_NW, _L), jnp.float32),
        compiler_params=pltpu.CompilerParams(needs_layout_passes=False),
        scratch_types=[
            pltpu.VMEM((2 * R, V), jnp.float32),
            pltpu.VMEM((2 * R, S), jnp.int32),
            pltpu.VMEM((RPW + _L,), jnp.float32),
            pltpu.VMEM((V + 8,), jnp.int32),
            pltpu.VMEM((_L,), jnp.float32),
            pltpu.SemaphoreType.DMA,
            pltpu.SemaphoreType.DMA,
        ],
    )
    def sc_loss(pred_hbm, tgt_hbm, w_hbm, out_hbm,
                pred_v, tgt_v, w_v, mark_v, out_v, sem_a, sem_b):
        wid = lax.axis_index("s") * _NC + lax.axis_index("c")
        row0 = wid * RPW
        lanes = lax.iota(jnp.int32, _L)
        tailf = (jnp.where(lanes >= (_L - REM), 1.0, 0.0)
                 if REM else jnp.zeros((_L,), jnp.float32))

        def start_chunk(g, boff, sem):
            base = row0 + g * R
            pltpu.async_copy(pred_hbm.at[pl.ds(base, R), :],
                             pred_v.at[pl.ds(boff, R), :], sem)
            pltpu.async_copy(tgt_hbm.at[pl.ds(base, R), :],
                             tgt_v.at[pl.ds(boff, R), :], sem)

        def wait_chunk(boff, sem):
            pltpu.make_async_copy(pred_hbm.at[pl.ds(0, R), :],
                                  pred_v.at[pl.ds(boff, R), :], sem).wait()
            pltpu.make_async_copy(tgt_hbm.at[pl.ds(0, R), :],
                                  tgt_v.at[pl.ds(boff, R), :], sem).wait()

        neg1 = jnp.full((_L,), -1, jnp.int32)

        start_chunk(0, 0, sem_a)
        start_chunk(1, R, sem_b)

        def init_body(k, _):
            mark_v[pl.ds(k * _L, _L)] = neg1
            return 0

        lax.fori_loop(0, (V + 8) // _L, init_body, 0)

        pltpu.sync_copy(w_hbm.at[pl.ds(row0, RPW)], w_v.at[pl.ds(0, RPW)])

        def process(g, boff, sem, acc):
            wait_chunk(boff, sem)

            def row_body(r, acc2):
                mse_a, set_a = acc2

                zero = jnp.zeros((_L,), jnp.float32)
                s2v = [zero, zero]
                sev = [zero, zero]
                for k in range(NFULL):
                    p = pred_v[boff + r, pl.ds(k * _L, _L)]
                    s2v[k % 2] = s2v[k % 2] + p * p
                    sev[k % 2] = sev[k % 2] + jnp.exp(p)
                pt = pred_v[boff + r, pl.ds(TAIL, _L)]
                s2v_t = s2v[0] + s2v[1] + tailf * pt * pt
                sev_t = sev[0] + sev[1] + tailf * jnp.exp(pt)

                mbase = (g * R + r) * (NJ * _L)
                for j in range(NJ):
                    t = tgt_v[boff + r, pl.ds(OFFS[j], _L)]
                    plsc.store_scatter(mark_v, [t], mbase + j * _L + lanes)
                npos_i = jnp.zeros((_L,), jnp.int32)
                sg_v = zero
                spe_v = zero
                sne_v = zero
                rfull = jnp.full((_L,), boff + r, jnp.int32)
                for j in range(NJ):
                    t = tgt_v[boff + r, pl.ds(OFFS[j], _L)]
                    rb = plsc.load_gather(mark_v, [t])
                    kb = rb == mbase + j * _L + lanes
                    keep = jnp.where(kb, 1.0, 0.0)
                    g_j = plsc.load_gather(pred_v, [rfull, t])
                    npos_i = npos_i + plsc.all_reduce_population_count(kb)
                    sg_v = sg_v + keep * g_j
                    spe_v = spe_v + keep * jnp.exp(-g_j)
                    sne_v = sne_v + keep * jnp.exp(g_j)
                npos = npos_i.astype(jnp.float32)
                bv = lambda x: jnp.full((_L,), x, jnp.float32)
                mseg = bv(jnp.sum(s2v_t - 2.0 * sg_v))
                se = bv(jnp.sum(sev_t))
                spe = bv(jnp.sum(spe_v))
                sne = bv(jnp.sum(sne_v))
                w = bv(w_v[pl.ds(g * R + r, _L)][0])
                mse_r = w * (mseg + npos)
                set_r = spe * (se - sne) / (npos * (float(V) - npos))
                return (mse_a + mse_r, set_a + set_r)

            acc = lax.fori_loop(0, R, row_body, acc, unroll=2)

            @pl.when(g + 2 < NCHUNK)
            def _():
                start_chunk(g + 2, boff, sem)

            return acc

        def pair_body(i, acc):
            g = i * 2
            acc = process(g, 0, sem_a, acc)
            acc = process(g + 1, R, sem_b, acc)
            return acc

        zv = jnp.zeros((_L,), jnp.float32)
        mse_acc, set_acc = lax.fori_loop(0, NCHUNK // 2, pair_body, (zv, zv))
        total = mse_acc * (1.0 / B) + (_LAMBDA / B) * set_acc
        out_v[...] = jnp.where(lanes == 0, total, 0.0)
        pltpu.sync_copy(out_v, out_hbm.at[wid])

    return sc_loss


def kernel(pred, target, weights):
    B, V = pred.shape
    S = target.shape[1]
    partials = _make_sc_loss(B, V, S, R=32)(pred, target, weights)
    return jnp.sum(partials)


# speedup vs baseline: 2.2387x; 1.0005x over previous
import functools

import jax
import jax.numpy as jnp
from jax import lax
from jax.experimental import pallas as pl
from jax.experimental.pallas import tpu as pltpu
from jax.experimental.pallas import tpu_sc as plsc

_LAMBDA = 10.0
_NC = 2
_NS = 16
_NW = _NC * _NS
_L = 16


def _make_sc_loss(B, V, S, R):
    assert B % (_NW * 2 * R) == 0 and V % 8 == 0 and S >= _L
    RPW = B // _NW
    NCHUNK = RPW // R
    NFULL = V // _L
    REM = V - NFULL * _L
    TAIL = V - _L
    OFFS = [k * _L for k in range(S // _L)] + ([S - _L] if S % _L else [])
    NJ = len(OFFS)

    mesh = plsc.VectorSubcoreMesh(core_axis_name="c", subcore_axis_name="s")

    @functools.partial(
        pl.kernel,
        mesh=mesh,
        out_type=jax.ShapeDtypeStruct((_NW, _L), jnp.float32),
        compiler_params=pltpu.CompilerParams(
            needs_layout_passes=False, use_tc_tiling_on_sc=True
        ),
        scratch_types=[
            pltpu.VMEM((2 * R, V), jnp.float32),
            pltpu.VMEM((2 * R, S), jnp.int32),
            pltpu.VMEM((RPW + _L,), jnp.float32),
            pltpu.VMEM((V + 8,), jnp.int32),
            pltpu.VMEM((_L,), jnp.float32),
            pltpu.SemaphoreType.DMA,
            pltpu.SemaphoreType.DMA,
        ],
    )
    def sc_loss(pred_hbm, tgt_hbm, w_hbm, out_hbm,
                pred_v, tgt_v, w_v, mark_v, out_v, sem_a, sem_b):
        wid = lax.axis_index("s") * _NC + lax.axis_index("c")
        row0 = wid * RPW
        lanes = lax.iota(jnp.int32, _L)
        tailf = (jnp.where(lanes >= (_L - REM), 1.0, 0.0)
                 if REM else jnp.zeros((_L,), jnp.float32))

        def start_chunk(g, boff, sem):
            base = row0 + g * R
            pltpu.async_copy(pred_hbm.at[pl.ds(base, R), :],
                             pred_v.at[pl.ds(boff, R), :], sem)
            pltpu.async_copy(tgt_hbm.at[pl.ds(base, R), :],
                             tgt_v.at[pl.ds(boff, R), :], sem)

        def wait_chunk(boff, sem):
            pltpu.make_async_copy(pred_hbm.at[pl.ds(0, R), :],
                                  pred_v.at[pl.ds(boff, R), :], sem).wait()
            pltpu.make_async_copy(tgt_hbm.at[pl.ds(0, R), :],
                                  tgt_v.at[pl.ds(boff, R), :], sem).wait()

        neg1 = jnp.full((_L,), -1, jnp.int32)

        start_chunk(0, 0, sem_a)
        start_chunk(1, R, sem_b)

        def init_body(k, _):
            mark_v[pl.ds(k * _L, _L)] = neg1
            return 0

        lax.fori_loop(0, (V + 8) // _L, init_body, 0)

        pltpu.sync_copy(w_hbm.at[pl.ds(row0, RPW)], w_v.at[pl.ds(0, RPW)])

        def process(g, boff, sem, acc):
            wait_chunk(boff, sem)

            def row_body(r, acc2):
                mse_a, set_a = acc2

                zero = jnp.zeros((_L,), jnp.float32)
                s2v = [zero, zero]
                sev = [zero, zero]
                for k in range(NFULL):
                    p = pred_v[boff + r, pl.ds(k * _L, _L)]
                    s2v[k % 2] = s2v[k % 2] + p * p
                    sev[k % 2] = sev[k % 2] + jnp.exp(p)
                pt = pred_v[boff + r, pl.ds(TAIL, _L)]
                s2v_t = s2v[0] + s2v[1] + tailf * pt * pt
                sev_t = sev[0] + sev[1] + tailf * jnp.exp(pt)

                mbase = (g * R + r) * (NJ * _L)
                for j in range(NJ):
                    t = tgt_v[boff + r, pl.ds(OFFS[j], _L)]
                    plsc.store_scatter(mark_v, [t], mbase + j * _L + lanes)
                npos_i = jnp.zeros((_L,), jnp.int32)
                sg_v = zero
                spe_v = zero
                sne_v = zero
                rfull = jnp.full((_L,), boff + r, jnp.int32)
                for j in range(NJ):
                    t = tgt_v[boff + r, pl.ds(OFFS[j], _L)]
                    rb = plsc.load_gather(mark_v, [t])
                    kb = rb == mbase + j * _L + lanes
                    keep = jnp.where(kb, 1.0, 0.0)
                    g_j = plsc.load_gather(pred_v, [rfull, t])
                    npos_i = npos_i + plsc.all_reduce_population_count(kb)
                    sg_v = sg_v + keep * g_j
                    spe_v = spe_v + keep * jnp.exp(-g_j)
                    sne_v = sne_v + keep * jnp.exp(g_j)
                npos = npos_i.astype(jnp.float32)
                bv = lambda x: jnp.full((_L,), x, jnp.float32)
                mseg = bv(jnp.sum(s2v_t - 2.0 * sg_v))
                se = bv(jnp.sum(sev_t))
                spe = bv(jnp.sum(spe_v))
                sne = bv(jnp.sum(sne_v))
                w = bv(w_v[pl.ds(g * R + r, _L)][0])
                mse_r = w * (mseg + npos)
                set_r = spe * (se - sne) / (npos * (float(V) - npos))
                return (mse_a + mse_r, set_a + set_r)

            acc = lax.fori_loop(0, R, row_body, acc, unroll=2)

            @pl.when(g + 2 < NCHUNK)
            def _():
                start_chunk(g + 2, boff, sem)

            return acc

        def pair_body(i, acc):
            g = i * 2
            acc = process(g, 0, sem_a, acc)
            acc = process(g + 1, R, sem_b, acc)
            return acc

        zv = jnp.zeros((_L,), jnp.float32)
        mse_acc, set_acc = lax.fori_loop(0, NCHUNK // 2, pair_body, (zv, zv))
        total = mse_acc * (1.0 / B) + (_LAMBDA / B) * set_acc
        out_v[...] = jnp.where(lanes == 0, total, 0.0)
        pltpu.sync_copy(out_v, out_hbm.at[wid])

    return sc_loss


def kernel(pred, target, weights):
    B, V = pred.shape
    S = target.shape[1]
    partials = _make_sc_loss(B, V, S, R=32)(pred, target, weights)
    return jnp.sum(partials)


# speedup vs baseline: 2.3167x; 1.0348x over previous
import functools

import jax
import jax.numpy as jnp
from jax import lax
from jax.experimental import pallas as pl
from jax.experimental.pallas import tpu as pltpu
from jax.experimental.pallas import tpu_sc as plsc

_LAMBDA = 10.0
_NC = 2
_NS = 16
_NW = _NC * _NS
_L = 16


def _make_sc_gather(B, V, S, R):
    assert B % (_NW * 2 * R) == 0 and R % _L == 0 and S >= _L
    RPW = B // _NW
    NCHUNK = RPW // R
    OFFS = [k * _L for k in range(S // _L)] + ([S - _L] if S % _L else [])
    NJ = len(OFFS)

    mesh = plsc.VectorSubcoreMesh(core_axis_name="c", subcore_axis_name="s")

    @functools.partial(
        pl.kernel,
        mesh=mesh,
        out_type=[jax.ShapeDtypeStruct((B,), jnp.float32)] * 4,
        compiler_params=pltpu.CompilerParams(
            needs_layout_passes=False, use_tc_tiling_on_sc=True
        ),
        scratch_types=[
            pltpu.VMEM((2 * R, V), jnp.float32),
            pltpu.VMEM((2 * R, S), jnp.int32),
            pltpu.VMEM((V + 8,), jnp.int32),
            pltpu.VMEM((RPW,), jnp.float32),
            pltpu.VMEM((RPW,), jnp.float32),
            pltpu.VMEM((RPW,), jnp.float32),
            pltpu.VMEM((RPW,), jnp.float32),
            pltpu.SemaphoreType.DMA,
            pltpu.SemaphoreType.DMA,
        ],
    )
    def sc_gather(pred_hbm, tgt_hbm, sg_hbm, spe_hbm, sne_hbm, np_hbm,
                  pred_v, tgt_v, mark_v, sg_v, spe_v, sne_v, np_v,
                  sem_a, sem_b):
        wid = lax.axis_index("s") * _NC + lax.axis_index("c")
        row0 = wid * RPW
        lanes = lax.iota(jnp.int32, _L)

        def start_chunk(g, boff, sem):
            base = row0 + g * R
            pltpu.async_copy(pred_hbm.at[pl.ds(base, R), :],
                             pred_v.at[pl.ds(boff, R), :], sem)
            pltpu.async_copy(tgt_hbm.at[pl.ds(base, R), :],
                             tgt_v.at[pl.ds(boff, R), :], sem)

        def wait_chunk(boff, sem):
            pltpu.make_async_copy(pred_hbm.at[pl.ds(0, R), :],
                                  pred_v.at[pl.ds(boff, R), :], sem).wait()
            pltpu.make_async_copy(tgt_hbm.at[pl.ds(0, R), :],
                                  tgt_v.at[pl.ds(boff, R), :], sem).wait()

        neg1 = jnp.full((_L,), -1, jnp.int32)

        start_chunk(0, 0, sem_a)
        start_chunk(1, R, sem_b)

        def init_body(k, _):
            mark_v[pl.ds(k * _L, _L)] = neg1
            return 0

        lax.fori_loop(0, (V + 8) // _L, init_body, 0)

        def process(g, boff, sem, packs):
            wait_chunk(boff, sem)

            def row_body(r, pk):
                psg, pspe, psne, pnp = pk
                zero = jnp.zeros((_L,), jnp.float32)

                mbase = (g * R + r) * (NJ * _L)
                for j in range(NJ):
                    t = tgt_v[boff + r, pl.ds(OFFS[j], _L)]
                    plsc.store_scatter(mark_v, [t], mbase + j * _L + lanes)
                npos_i = jnp.zeros((_L,), jnp.int32)
                sg_acc = zero
                spe_acc = zero
                sne_acc = zero
                rfull = jnp.full((_L,), boff + r, jnp.int32)
                for j in range(NJ):
                    t = tgt_v[boff + r, pl.ds(OFFS[j], _L)]
                    rb = plsc.load_gather(mark_v, [t])
                    kb = rb == mbase + j * _L + lanes
                    keep = jnp.where(kb, 1.0, 0.0)
                    g_j = plsc.load_gather(pred_v, [rfull, t])
                    npos_i = npos_i + plsc.all_reduce_population_count(kb)
                    sg_acc = sg_acc + keep * g_j
                    spe_acc = spe_acc + keep * jnp.exp(-g_j)
                    sne_acc = sne_acc + keep * jnp.exp(g_j)

                bv = lambda x: jnp.full((_L,), x, jnp.float32)
                lm = lanes == r % _L
                psg = jnp.where(lm, bv(jnp.sum(sg_acc)), psg)
                pspe = jnp.where(lm, bv(jnp.sum(spe_acc)), pspe)
                psne = jnp.where(lm, bv(jnp.sum(sne_acc)), psne)
                pnp = jnp.where(lm, npos_i.astype(jnp.float32), pnp)

                @pl.when(r % _L == _L - 1)
                def _():
                    off = g * R + r - (_L - 1)
                    sg_v[pl.ds(off, _L)] = psg
                    spe_v[pl.ds(off, _L)] = pspe
                    sne_v[pl.ds(off, _L)] = psne
                    np_v[pl.ds(off, _L)] = pnp

                return (psg, pspe, psne, pnp)

            packs = lax.fori_loop(0, R, row_body, packs, unroll=2)

            @pl.when(g + 2 < NCHUNK)
            def _():
                start_chunk(g + 2, boff, sem)

            return packs

        def pair_body(i, packs):
            g = i * 2
            packs = process(g, 0, sem_a, packs)
            packs = process(g + 1, R, sem_b, packs)
            return packs

        zv = jnp.zeros((_L,), jnp.float32)
        lax.fori_loop(0, NCHUNK // 2, pair_body, (zv, zv, zv, zv))

        pltpu.sync_copy(sg_v, sg_hbm.at[pl.ds(row0, RPW)])
        pltpu.sync_copy(spe_v, spe_hbm.at[pl.ds(row0, RPW)])
        pltpu.sync_copy(sne_v, sne_hbm.at[pl.ds(row0, RPW)])
        pltpu.sync_copy(np_v, np_hbm.at[pl.ds(row0, RPW)])

    return sc_gather


def _tc_dense(pred, blk):
    B, V = pred.shape

    def kern(p_ref, s2_ref, se_ref):
        p = p_ref[...]
        s2_ref[...] = jnp.sum(p * p, axis=1)
        se_ref[...] = jnp.sum(jnp.exp(p), axis=1)

    return pl.pallas_call(
        kern,
        grid=(B // blk,),
        in_specs=[pl.BlockSpec((blk, V), lambda i: (i, 0))],
        out_specs=[pl.BlockSpec((blk,), lambda i: (i,)),
                   pl.BlockSpec((blk,), lambda i: (i,))],
        out_shape=[jax.ShapeDtypeStruct((B,), jnp.float32)] * 2,
    )(pred)


def _tc_combine(s2, se, sg, spe, sne, npos, w, B, V):
    def kern(s2_ref, se_ref, sg_ref, spe_ref, sne_ref, np_ref, w_ref, o_ref):
        npos = np_ref[...]
        mse = w_ref[...] * (s2_ref[...] - 2.0 * sg_ref[...] + npos)
        sett = spe_ref[...] * (se_ref[...] - sne_ref[...]) / (
            npos * (float(V) - npos))
        total = (jnp.sum(mse) + _LAMBDA * jnp.sum(sett)) * (1.0 / B)
        o_ref[...] = jnp.reshape(total, (1, 1))

    return pl.pallas_call(
        kern,
        out_shape=jax.ShapeDtypeStruct((1, 1), jnp.float32),
    )(s2, se, sg, spe, sne, npos, w)


def kernel(pred, target, weights):
    B, V = pred.shape
    S = target.shape[1]
    sg, spe, sne, npos = _make_sc_gather(B, V, S, R=32)(pred, target)
    s2, se = _tc_dense(pred, blk=512)
    out = _tc_combine(s2, se, sg, spe, sne, npos, weights, B, V)
    return out[0, 0]
